# Initial kernel scaffold; baseline (speedup 1.0000x reference)
import jax
import jax.numpy as jnp
from jax.experimental import pallas as pl


def kernel(input_ids, token_type_ids, word_emb, type_emb, ln_weight):
    raise NotImplementedError("write your pallas kernel here")


# speedup vs baseline: 1.5601x; 1.5601x over previous
import functools

import jax
import jax.numpy as jnp
from jax import lax
from jax.experimental import pallas as pl
from jax.experimental.pallas import tpu as pltpu
from jax.experimental.pallas import tpu_sc as plsc

HIDDEN = 128
B, T = 4, 2048
EPS = 1e-6
NTOK = B * T
NW = 32
ROWS_PER_W = NTOK // NW
L = 16
NCH = HIDDEN // L
GCH = 128
NG = ROWS_PER_W // GCH


def _rsqrt16(x):
    i = lax.bitcast_convert_type(x, jnp.int32)
    i = jnp.int32(0x5F3759DF) - lax.shift_right_arithmetic(i, 1)
    y = lax.bitcast_convert_type(i, jnp.float32)
    xh = x * 0.5
    for _ in range(3):
        y = y * (1.5 - xh * y * y)
    return y


@functools.partial(
    pl.kernel,
    out_type=jax.ShapeDtypeStruct((NTOK, HIDDEN), jnp.float32),
    mesh=plsc.VectorSubcoreMesh(core_axis_name="c", subcore_axis_name="s"),
    scratch_types=[
        pltpu.VMEM((NG, GCH), jnp.int32),
        pltpu.VMEM((ROWS_PER_W,), jnp.int32),
        pltpu.VMEM((2 * HIDDEN,), jnp.float32),
        pltpu.VMEM((HIDDEN,), jnp.float32),
        pltpu.VMEM((ROWS_PER_W, HIDDEN), jnp.float32),
        pltpu.SemaphoreType.DMA,
        pltpu.SemaphoreType.DMA,
    ],
)
def _emb_kernel(word_hbm, ids_hbm, tt_hbm, type_hbm, w_hbm, out_hbm,
                idx_v, tt_v, type_v, w_v, rows_v, sem0, sem1):
    wid = lax.axis_index("s") * 2 + lax.axis_index("c")
    base = wid * ROWS_PER_W

    for g in range(NG):
        pltpu.sync_copy(ids_hbm.at[pl.ds(base + g * GCH, GCH)], idx_v.at[g])
    pltpu.sync_copy(tt_hbm.at[pl.ds(base, ROWS_PER_W)], tt_v)
    pltpu.sync_copy(type_hbm, type_v)
    pltpu.sync_copy(w_hbm, w_v)

    sems = (sem0, sem1)
    copies = []
    for g in range(NG):
        copies.append(pltpu.async_copy(
            word_hbm.at[idx_v.at[g]],
            rows_v.at[pl.ds(g * GCH, GCH)],
            sems[g % 2],
        ))
    for c in copies:
        c.wait()

    t0 = [type_v[pl.ds(c * L, L)] for c in range(NCH)]
    td = [type_v[pl.ds(HIDDEN + c * L, L)] - t0[c] for c in range(NCH)]
    wch = [w_v[pl.ds(c * L, L)] for c in range(NCH)]

    lanes = lax.iota(jnp.int32, L)
    perms = [lax.bitwise_xor(lanes, jnp.int32(k)) for k in (1, 2, 4, 8)]

    def group_body(g, carry):
        rbase = g * L
        ttf16 = tt_v[pl.ds(rbase, L)].astype(jnp.float32)
        for rr in range(L):
            r = rbase + rr
            ttf = jnp.broadcast_to(ttf16[rr], (L,))
            xs = []
            acc0 = jnp.zeros((L,), jnp.float32)
            acc1 = jnp.zeros((L,), jnp.float32)
            for c in range(NCH):
                xc = rows_v[r, pl.ds(c * L, L)] + (t0[c] + ttf * td[c])
                xs.append(xc)
                if c % 2 == 0:
                    acc0 = acc0 + xc * xc
                else:
                    acc1 = acc1 + xc * xc
            s = acc0 + acc1
            for p in perms:
                s = s + s.at[p].get(mode="promise_in_bounds")
            var = s * (1.0 / HIDDEN) + EPS
            scale = _rsqrt16(var)
            for c in range(NCH):
                rows_v[r, pl.ds(c * L, L)] = (xs[c] * scale) * wch[c]
        return carry

    lax.fori_loop(0, ROWS_PER_W // L, group_body, 0)

    pltpu.sync_copy(rows_v, out_hbm.at[pl.ds(base, ROWS_PER_W)])


def kernel(input_ids, token_type_ids, word_emb, type_emb, ln_weight):
    ids = input_ids.reshape(-1).astype(jnp.int32)
    tt = token_type_ids.reshape(-1).astype(jnp.int32)
    out = _emb_kernel(word_emb, ids, tt, type_emb.reshape(-1), ln_weight)
    return out.reshape(B, T, HIDDEN)
